# Initial kernel scaffold; baseline (speedup 1.0000x reference)
import jax
import jax.numpy as jnp
from jax.experimental import pallas as pl


def kernel(tokenIndex, weights):
    raise NotImplementedError("write your pallas kernel here")


# speedup vs baseline: 1.5752x; 1.5752x over previous
import functools

import jax
import jax.numpy as jnp
from jax import lax
from jax.experimental import pallas as pl
from jax.experimental.pallas import tpu as pltpu
from jax.experimental.pallas import tpu_sc as plsc

_VOCAB = 1_000_000
_EMBED = 32
_BATCH = 16384
_FIELDS = 26
_TOTAL = _BATCH * _FIELDS

_info = plsc.get_sparse_core_info()
_NC = _info.num_cores
_NS = _info.num_subcores
_NW = _NC * _NS
_PER_W = _TOTAL // _NW
_CHUNK = 1024
_NCHUNKS = _PER_W // _CHUNK

_mesh = plsc.VectorSubcoreMesh(core_axis_name="c", subcore_axis_name="s")


@functools.partial(
    pl.kernel,
    mesh=_mesh,
    out_type=jax.ShapeDtypeStruct((_TOTAL, _EMBED), jnp.float32),
    scratch_types=[
        pltpu.VMEM((_NCHUNKS, _CHUNK), jnp.int32),
        pltpu.VMEM((_CHUNK, _EMBED), jnp.float32),
        pltpu.VMEM((_CHUNK, _EMBED), jnp.float32),
        pltpu.SemaphoreType.DMA,
        pltpu.SemaphoreType.DMA,
        pltpu.SemaphoreType.DMA,
        pltpu.SemaphoreType.DMA,
    ],
    compiler_params=pltpu.CompilerParams(use_tc_tiling_on_sc=False),
)
def _gather_all(table_hbm, idx_hbm, out_hbm, idx_v, rows0, rows1,
                gsem0, gsem1, ssem0, ssem1):
    wid = lax.axis_index("s") * _NC + lax.axis_index("c")
    base = wid * _PER_W
    bufs = (rows0, rows1)
    gsems = (gsem0, gsem1)
    ssems = (ssem0, ssem1)

    pltpu.sync_copy(idx_hbm.at[wid], idx_v)

    def gstart(j):
        return pltpu.async_copy(
            table_hbm.at[idx_v.at[j]], bufs[j % 2], gsems[j % 2])

    def sstart(j):
        return pltpu.async_copy(
            bufs[j % 2], out_hbm.at[pl.ds(base + j * _CHUNK, _CHUNK)],
            ssems[j % 2])

    g = [None] * _NCHUNKS
    s = [None] * _NCHUNKS
    g[0] = gstart(0)
    for j in range(_NCHUNKS):
        if j + 1 < _NCHUNKS:
            if j >= 1:
                s[j - 1].wait()
            g[j + 1] = gstart(j + 1)
        g[j].wait()
        s[j] = sstart(j)
    if _NCHUNKS >= 2:
        s[_NCHUNKS - 2].wait()
    s[_NCHUNKS - 1].wait()


def kernel(tokenIndex, weights):
    idx = tokenIndex.reshape(_NW, _NCHUNKS, _CHUNK).astype(jnp.int32)
    out = _gather_all(weights, idx)
    return out.reshape(_BATCH, _FIELDS, _EMBED)
